# baseline (speedup 1.0000x reference)
import functools
import math

import jax
import jax.numpy as jnp
from jax.experimental import pallas as pl


def _fma_kernel(x_ref, p_ref, o_ref, *, scale):
    o_ref[...] = x_ref[...] * scale + p_ref[...]


def kernel(x, pos_emb):
    B, L, D = x.shape
    scale = math.sqrt(D)
    LD = L * D
    xt = x.transpose(1, 2, 0).reshape(LD, B)
    pf = pos_emb.reshape(LD, 1)
    blk = 256
    out = pl.pallas_call(
        functools.partial(_fma_kernel, scale=scale),
        grid=(LD // blk,),
        in_specs=[
            pl.BlockSpec((blk, B), lambda i: (i, 0)),
            pl.BlockSpec((blk, 1), lambda i: (i, 0)),
        ],
        out_specs=pl.BlockSpec((blk, B), lambda i: (i, 0)),
        out_shape=jax.ShapeDtypeStruct((LD, B), x.dtype),
    )(xt, pf)
    return out.reshape(L, D, B).transpose(2, 0, 1)


# speedup vs baseline: 1.0897x; 1.0897x over previous
import functools
import math

import jax
import jax.numpy as jnp
from jax.experimental import pallas as pl


def _fma_kernel(x_ref, p_ref, o_ref, *, scale):
    o_ref[...] = x_ref[...] * scale + p_ref[...][:, :, None]


def kernel(x, pos_emb):
    B, L, D = x.shape
    scale = math.sqrt(D)
    xt = x.transpose(1, 2, 0)
    lblk = 8
    out = pl.pallas_call(
        functools.partial(_fma_kernel, scale=scale),
        grid=(L // lblk,),
        in_specs=[
            pl.BlockSpec((lblk, D, B), lambda i: (i, 0, 0)),
            pl.BlockSpec((lblk, D), lambda i: (i, 0)),
        ],
        out_specs=pl.BlockSpec((lblk, D, B), lambda i: (i, 0, 0)),
        out_shape=jax.ShapeDtypeStruct((L, D, B), x.dtype),
    )(xt, pos_emb)
    return out.transpose(2, 0, 1)


# speedup vs baseline: 1.1011x; 1.0104x over previous
import functools
import math

import jax
import jax.numpy as jnp
from jax.experimental import pallas as pl
from jax.experimental.pallas import tpu as pltpu


def _fma_kernel(p_nat_ref, x_ref, o_ref, p_scr, *, scale, lblk):
    @pl.when(pl.program_id(0) == 0)
    def _():
        p_scr[...] = jnp.transpose(p_nat_ref[...], (1, 0))

    i = pl.program_id(0)
    p = p_scr[pl.ds(i * lblk, lblk), :]
    o_ref[...] = x_ref[...] * scale + p[:, :, None]


def kernel(x, pos_emb):
    B, L, D = x.shape
    scale = math.sqrt(D)
    xt = x.transpose(1, 2, 0)
    p_nat = pos_emb.transpose(1, 0)
    lblk = 8
    out = pl.pallas_call(
        functools.partial(_fma_kernel, scale=scale, lblk=lblk),
        grid=(L // lblk,),
        in_specs=[
            pl.BlockSpec((D, L), lambda i: (0, 0)),
            pl.BlockSpec((lblk, D, B), lambda i: (i, 0, 0)),
        ],
        out_specs=pl.BlockSpec((lblk, D, B), lambda i: (i, 0, 0)),
        out_shape=jax.ShapeDtypeStruct((L, D, B), x.dtype),
        scratch_shapes=[pltpu.VMEM((L, D), jnp.float32)],
    )(p_nat, xt)
    return out.transpose(2, 0, 1)
